# Initial kernel scaffold; baseline (speedup 1.0000x reference)
import functools

import jax
import jax.numpy as jnp
from jax import lax
from jax.experimental import pallas as pl
from jax.experimental.pallas import tpu as pltpu
from jax.experimental.pallas import tpu_sc as plsc

N_NODES = 10000
N_EDGES = 320000
D_FEAT = 128

NUM_CORES = 2
NUM_SUBCORES = 16
NUM_WORKERS = NUM_CORES * NUM_SUBCORES
EDGES_PER_WORKER = N_EDGES // NUM_WORKERS
CHUNK = 80
NUM_CHUNKS = EDGES_PER_WORKER // CHUNK
ROWS_PER_TILE = N_NODES // NUM_SUBCORES


def _sc_layer(x_hbm, src_hbm, dst_hbm, zeros_hbm, out_hbm,
              sidx, didx, rows, acc, sem):
    cid = lax.axis_index("c")
    sid = lax.axis_index("s")
    wid = sid * NUM_CORES + cid

    row0 = sid * ROWS_PER_TILE
    pltpu.sync_copy(zeros_hbm.at[pl.ds(row0, ROWS_PER_TILE)],
                    acc.at[pl.ds(row0, ROWS_PER_TILE)])
    plsc.subcore_barrier()

    base = wid * EDGES_PER_WORKER

    def body(c, _):
        off = base + c * CHUNK
        pltpu.sync_copy(src_hbm.at[pl.ds(off, CHUNK)], sidx)
        pltpu.sync_copy(dst_hbm.at[pl.ds(off, CHUNK)], didx)
        pltpu.async_copy(x_hbm.at[sidx], rows, sem).wait()
        pltpu.sync_copy(rows, acc.at[didx], add=True)
        return ()

    lax.fori_loop(0, NUM_CHUNKS, body, ())
    plsc.subcore_barrier()

    pltpu.sync_copy(acc.at[pl.ds(row0, ROWS_PER_TILE)],
                    out_hbm.at[cid, pl.ds(row0, ROWS_PER_TILE)])


_layer_call = functools.partial(
    pl.kernel,
    out_type=jax.ShapeDtypeStruct((NUM_CORES, N_NODES, D_FEAT), jnp.float32),
    mesh=plsc.VectorSubcoreMesh(core_axis_name="c", subcore_axis_name="s"),
    scratch_types=[
        pltpu.VMEM((CHUNK,), jnp.int32),
        pltpu.VMEM((CHUNK,), jnp.int32),
        pltpu.VMEM((CHUNK, D_FEAT), jnp.float32),
        pltpu.VMEM_SHARED((N_NODES, D_FEAT), jnp.float32),
        pltpu.SemaphoreType.DMA,
    ],
)(_sc_layer)


def _combine_body(p_ref, o_ref, *, relu):
    s = p_ref[0] + p_ref[1]
    o_ref[...] = jnp.maximum(s, 0.0) if relu else s


def _combine(partials, relu):
    blk = 2000
    return pl.pallas_call(
        functools.partial(_combine_body, relu=relu),
        out_shape=jax.ShapeDtypeStruct((N_NODES, D_FEAT), jnp.float32),
        grid=(N_NODES // blk,),
        in_specs=[pl.BlockSpec((NUM_CORES, blk, D_FEAT), lambda i: (0, i, 0))],
        out_specs=pl.BlockSpec((blk, D_FEAT), lambda i: (i, 0)),
    )(partials)


def kernel(x, edge_index):
    src = edge_index[0].astype(jnp.int32)
    dst = edge_index[1].astype(jnp.int32)
    zeros = jnp.zeros((N_NODES, D_FEAT), jnp.float32)

    p1 = _layer_call(x, src, dst, zeros)
    h = _combine(p1, relu=True)
    p2 = _layer_call(h, src, dst, zeros)
    return _combine(p2, relu=False)


# speedup vs baseline: 5.1938x; 5.1938x over previous
import functools

import jax
import jax.numpy as jnp
from jax import lax
from jax.experimental import pallas as pl
from jax.experimental.pallas import tpu as pltpu
from jax.experimental.pallas import tpu_sc as plsc

N_NODES = 10000
N_EDGES = 320000
D_FEAT = 128

NUM_CORES = 2
NUM_SUBCORES = 16
NUM_WORKERS = NUM_CORES * NUM_SUBCORES
EDGES_PER_WORKER = N_EDGES // NUM_WORKERS
CHUNK = 80
NUM_CHUNKS = EDGES_PER_WORKER // CHUNK
ROWS_PER_TILE = 624
TAIL_ROWS = N_NODES - ROWS_PER_TILE * NUM_SUBCORES
TAIL0 = ROWS_PER_TILE * NUM_SUBCORES


def _sc_layer(x_hbm, src_hbm, dst_hbm, zeros_hbm, out_hbm,
              sidx, didx, rows, acc, sem):
    cid = lax.axis_index("c")
    sid = lax.axis_index("s")
    wid = sid * NUM_CORES + cid

    row0 = sid * ROWS_PER_TILE
    pltpu.sync_copy(zeros_hbm.at[pl.ds(row0, ROWS_PER_TILE)],
                    acc.at[pl.ds(row0, ROWS_PER_TILE)])

    @pl.when(sid == NUM_SUBCORES - 1)
    def _zero_tail():
        pltpu.sync_copy(zeros_hbm.at[pl.ds(TAIL0, TAIL_ROWS)],
                        acc.at[pl.ds(TAIL0, TAIL_ROWS)])

    plsc.subcore_barrier()

    base = wid * EDGES_PER_WORKER

    def body(c, _):
        off = base + c * CHUNK
        pltpu.sync_copy(src_hbm.at[pl.ds(off, CHUNK)], sidx)
        pltpu.sync_copy(dst_hbm.at[pl.ds(off, CHUNK)], didx)
        pltpu.async_copy(x_hbm.at[sidx], rows, sem).wait()
        pltpu.sync_copy(rows, acc.at[didx], add=True)
        return ()

    lax.fori_loop(0, NUM_CHUNKS, body, ())
    plsc.subcore_barrier()

    pltpu.sync_copy(acc.at[pl.ds(row0, ROWS_PER_TILE)],
                    out_hbm.at[cid, pl.ds(row0, ROWS_PER_TILE)])

    @pl.when(sid == NUM_SUBCORES - 1)
    def _out_tail():
        pltpu.sync_copy(acc.at[pl.ds(TAIL0, TAIL_ROWS)],
                        out_hbm.at[cid, pl.ds(TAIL0, TAIL_ROWS)])


_layer_call = functools.partial(
    pl.kernel,
    out_type=jax.ShapeDtypeStruct((NUM_CORES, N_NODES, D_FEAT), jnp.float32),
    mesh=plsc.VectorSubcoreMesh(core_axis_name="c", subcore_axis_name="s"),
    scratch_types=[
        pltpu.VMEM((CHUNK,), jnp.int32),
        pltpu.VMEM((CHUNK,), jnp.int32),
        pltpu.VMEM((CHUNK, D_FEAT), jnp.float32),
        pltpu.VMEM_SHARED((N_NODES, D_FEAT), jnp.float32),
        pltpu.SemaphoreType.DMA,
    ],
)(_sc_layer)


def _combine_body(p_ref, o_ref, *, relu):
    s = p_ref[0] + p_ref[1]
    o_ref[...] = jnp.maximum(s, 0.0) if relu else s


def _combine(partials, relu):
    blk = 2000
    return pl.pallas_call(
        functools.partial(_combine_body, relu=relu),
        out_shape=jax.ShapeDtypeStruct((N_NODES, D_FEAT), jnp.float32),
        grid=(N_NODES // blk,),
        in_specs=[pl.BlockSpec((NUM_CORES, blk, D_FEAT), lambda i: (0, i, 0))],
        out_specs=pl.BlockSpec((blk, D_FEAT), lambda i: (i, 0)),
    )(partials)


def kernel(x, edge_index):
    src = edge_index[0].astype(jnp.int32)
    dst = edge_index[1].astype(jnp.int32)
    zeros = jnp.zeros((N_NODES, D_FEAT), jnp.float32)

    p1 = _layer_call(x, src, dst, zeros)
    h = _combine(p1, relu=True)
    p2 = _layer_call(h, src, dst, zeros)
    return _combine(p2, relu=False)


# speedup vs baseline: 7.5102x; 1.4460x over previous
import functools

import jax
import jax.numpy as jnp
from jax import lax
from jax.experimental import pallas as pl
from jax.experimental.pallas import tpu as pltpu
from jax.experimental.pallas import tpu_sc as plsc

N_NODES = 10000
N_EDGES = 320000
D_FEAT = 128

NUM_CORES = 2
NUM_SUBCORES = 16
NUM_WORKERS = NUM_CORES * NUM_SUBCORES
EDGES_PER_WORKER = N_EDGES // NUM_WORKERS
CHUNK = 96
NUM_CHUNKS = -(-EDGES_PER_WORKER // CHUNK)
EPW_PAD = NUM_CHUNKS * CHUNK
TRASH_ROW = N_NODES
ACC_ROWS = N_NODES + 8
ROWS_PER_TILE = 624
TAIL_ROWS = N_NODES - ROWS_PER_TILE * NUM_SUBCORES
TAIL0 = ROWS_PER_TILE * NUM_SUBCORES
NSLOT = 3


def _sc_layer(x_hbm, src_hbm, dst_hbm, zeros_hbm, out_hbm,
              sidx, didx, rows, acc, isem, gsem, ssem):
    cid = lax.axis_index("c")
    sid = lax.axis_index("s")
    wid = sid * NUM_CORES + cid

    row0 = sid * ROWS_PER_TILE
    pltpu.sync_copy(zeros_hbm.at[pl.ds(row0, ROWS_PER_TILE)],
                    acc.at[pl.ds(row0, ROWS_PER_TILE)])

    @pl.when(sid == NUM_SUBCORES - 1)
    def _zero_tail():
        pltpu.sync_copy(zeros_hbm.at[pl.ds(TAIL0, TAIL_ROWS)],
                        acc.at[pl.ds(TAIL0, TAIL_ROWS)])

    plsc.subcore_barrier()

    base = wid * EPW_PAD

    def start_idx(c, s):
        off = base + c * CHUNK
        pltpu.async_copy(src_hbm.at[pl.ds(off, CHUNK)], sidx.at[s], isem.at[s])
        pltpu.async_copy(dst_hbm.at[pl.ds(off, CHUNK)], didx.at[s], isem.at[s])

    def wait_idx(c, s):
        off = base + c * CHUNK
        pltpu.make_async_copy(src_hbm.at[pl.ds(off, CHUNK)], sidx.at[s],
                              isem.at[s]).wait()
        pltpu.make_async_copy(dst_hbm.at[pl.ds(off, CHUNK)], didx.at[s],
                              isem.at[s]).wait()

    def start_gather(c, b):
        s = lax.rem(c, NSLOT)
        pltpu.async_copy(x_hbm.at[sidx.at[s]], rows.at[b], gsem.at[b])

    def wait_gather(c, b):
        s = lax.rem(c, NSLOT)
        pltpu.make_async_copy(x_hbm.at[sidx.at[s]], rows.at[b],
                              gsem.at[b]).wait()

    def start_scatter(c, b):
        s = lax.rem(c, NSLOT)
        pltpu.async_copy(rows.at[b], acc.at[didx.at[s]], ssem.at[b], add=True)

    def wait_scatter(c, b):
        s = lax.rem(c, NSLOT)
        pltpu.make_async_copy(rows.at[b], acc.at[didx.at[s]],
                              ssem.at[b]).wait()

    start_idx(0, 0)
    start_idx(1, 1)
    wait_idx(0, 0)
    start_gather(0, 0)
    start_idx(2, 2)
    wait_gather(0, 0)
    start_scatter(0, 0)
    wait_idx(1, 1)
    start_gather(1, 1)

    def body(c, _):
        b = lax.rem(c, 2)
        nb = 1 - b
        wait_scatter(c - 1, nb)

        @pl.when(c + 2 < NUM_CHUNKS)
        def _prefetch():
            start_idx(c + 2, lax.rem(c + 2, NSLOT))

        wait_idx(c + 1, lax.rem(c + 1, NSLOT))
        start_gather(c + 1, nb)
        wait_gather(c, b)
        start_scatter(c, b)
        return ()

    lax.fori_loop(1, NUM_CHUNKS - 1, body, ())

    last = NUM_CHUNKS - 1
    wait_scatter(last - 1, (last - 1) % 2)
    wait_gather(last, last % 2)
    start_scatter(last, last % 2)
    wait_scatter(last, last % 2)
    plsc.subcore_barrier()

    pltpu.sync_copy(acc.at[pl.ds(row0, ROWS_PER_TILE)],
                    out_hbm.at[cid, pl.ds(row0, ROWS_PER_TILE)])

    @pl.when(sid == NUM_SUBCORES - 1)
    def _out_tail():
        pltpu.sync_copy(acc.at[pl.ds(TAIL0, TAIL_ROWS)],
                        out_hbm.at[cid, pl.ds(TAIL0, TAIL_ROWS)])


_layer_call = functools.partial(
    pl.kernel,
    out_type=jax.ShapeDtypeStruct((NUM_CORES, N_NODES, D_FEAT), jnp.float32),
    mesh=plsc.VectorSubcoreMesh(core_axis_name="c", subcore_axis_name="s"),
    scratch_types=[
        pltpu.VMEM((NSLOT, CHUNK), jnp.int32),
        pltpu.VMEM((NSLOT, CHUNK), jnp.int32),
        pltpu.VMEM((2, CHUNK, D_FEAT), jnp.float32),
        pltpu.VMEM_SHARED((ACC_ROWS, D_FEAT), jnp.float32),
        pltpu.SemaphoreType.DMA((NSLOT,)),
        pltpu.SemaphoreType.DMA((2,)),
        pltpu.SemaphoreType.DMA((2,)),
    ],
)(_sc_layer)


def _combine_body(p_ref, o_ref, *, relu):
    s = p_ref[0] + p_ref[1]
    o_ref[...] = jnp.maximum(s, 0.0) if relu else s


def _combine(partials, relu):
    blk = 2000
    return pl.pallas_call(
        functools.partial(_combine_body, relu=relu),
        out_shape=jax.ShapeDtypeStruct((N_NODES, D_FEAT), jnp.float32),
        grid=(N_NODES // blk,),
        in_specs=[pl.BlockSpec((NUM_CORES, blk, D_FEAT), lambda i: (0, i, 0))],
        out_specs=pl.BlockSpec((blk, D_FEAT), lambda i: (i, 0)),
    )(partials)


def kernel(x, edge_index):
    src = edge_index[0].astype(jnp.int32).reshape(NUM_WORKERS, EDGES_PER_WORKER)
    dst = edge_index[1].astype(jnp.int32).reshape(NUM_WORKERS, EDGES_PER_WORKER)
    pad = EPW_PAD - EDGES_PER_WORKER
    src_pad = jnp.pad(src, ((0, 0), (0, pad))).reshape(-1)
    dst_pad = jnp.pad(dst, ((0, 0), (0, pad)),
                      constant_values=TRASH_ROW).reshape(-1)
    zeros = jnp.zeros((N_NODES, D_FEAT), jnp.float32)

    p1 = _layer_call(x, src_pad, dst_pad, zeros)
    h = _combine(p1, relu=True)
    p2 = _layer_call(h, src_pad, dst_pad, zeros)
    return _combine(p2, relu=False)


# speedup vs baseline: 9.6010x; 1.2784x over previous
import functools

import jax
import jax.numpy as jnp
from jax import lax
from jax.experimental import pallas as pl
from jax.experimental.pallas import tpu as pltpu
from jax.experimental.pallas import tpu_sc as plsc

N_NODES = 10000
N_EDGES = 320000
D_FEAT = 128

NUM_CORES = 2
NUM_SUBCORES = 16
NUM_WORKERS = NUM_CORES * NUM_SUBCORES
EDGES_PER_WORKER = N_EDGES // NUM_WORKERS
CHUNK = 64
NUM_CHUNKS = -(-EDGES_PER_WORKER // CHUNK)
EPW_PAD = NUM_CHUNKS * CHUNK
NBUF = 4
TRASH_ROW = N_NODES
ACC_ROWS = N_NODES + 8
ROWS_PER_TILE = 624
TAIL_ROWS = N_NODES - ROWS_PER_TILE * NUM_SUBCORES
TAIL0 = ROWS_PER_TILE * NUM_SUBCORES
NSLOT = 6


def _sc_layer(x_hbm, src_hbm, dst_hbm, zeros_hbm, out_hbm,
              sidx, didx, rows, acc, isem, gsem, ssem):
    cid = lax.axis_index("c")
    sid = lax.axis_index("s")
    wid = sid * NUM_CORES + cid

    row0 = sid * ROWS_PER_TILE
    pltpu.sync_copy(zeros_hbm.at[pl.ds(row0, ROWS_PER_TILE)],
                    acc.at[pl.ds(row0, ROWS_PER_TILE)])

    @pl.when(sid == NUM_SUBCORES - 1)
    def _zero_tail():
        pltpu.sync_copy(zeros_hbm.at[pl.ds(TAIL0, TAIL_ROWS)],
                        acc.at[pl.ds(TAIL0, TAIL_ROWS)])

    plsc.subcore_barrier()

    base = wid * EPW_PAD

    def start_idx(c, s):
        off = base + c * CHUNK
        pltpu.async_copy(src_hbm.at[pl.ds(off, CHUNK)], sidx.at[s], isem.at[s])
        pltpu.async_copy(dst_hbm.at[pl.ds(off, CHUNK)], didx.at[s], isem.at[s])

    def wait_idx(c, s):
        off = base + c * CHUNK
        pltpu.make_async_copy(src_hbm.at[pl.ds(off, CHUNK)], sidx.at[s],
                              isem.at[s]).wait()
        pltpu.make_async_copy(dst_hbm.at[pl.ds(off, CHUNK)], didx.at[s],
                              isem.at[s]).wait()

    def start_gather(c, b):
        s = lax.rem(c, NSLOT)
        pltpu.async_copy(x_hbm.at[sidx.at[s]], rows.at[b], gsem.at[b])

    def wait_gather(c, b):
        s = lax.rem(c, NSLOT)
        pltpu.make_async_copy(x_hbm.at[sidx.at[s]], rows.at[b],
                              gsem.at[b]).wait()

    def start_scatter(c, b):
        s = lax.rem(c, NSLOT)
        pltpu.async_copy(rows.at[b], acc.at[didx.at[s]], ssem.at[b], add=True)

    def wait_scatter(c, b):
        s = lax.rem(c, NSLOT)
        pltpu.make_async_copy(rows.at[b], acc.at[didx.at[s]],
                              ssem.at[b]).wait()

    for c in range(4):
        start_idx(c, c % NSLOT)
    wait_idx(0, 0)
    start_gather(0, 0)
    wait_idx(1, 1)
    start_gather(1, 1)

    def body(c, _):
        b = lax.rem(c, NBUF)

        @pl.when(c >= 2)
        def _drain():
            wait_scatter(c - 2, lax.rem(c - 2, NBUF))

        @pl.when(c + 4 < NUM_CHUNKS)
        def _prefetch():
            start_idx(c + 4, lax.rem(c + 4, NSLOT))

        wait_idx(c + 2, lax.rem(c + 2, NSLOT))
        start_gather(c + 2, lax.rem(c + 2, NBUF))
        wait_gather(c, b)
        start_scatter(c, b)
        return ()

    lax.fori_loop(0, NUM_CHUNKS - 2, body, ())

    for c in (NUM_CHUNKS - 2, NUM_CHUNKS - 1):
        wait_scatter(c - 2, (c - 2) % NBUF)
        wait_gather(c, c % NBUF)
        start_scatter(c, c % NBUF)
    wait_scatter(NUM_CHUNKS - 2, (NUM_CHUNKS - 2) % NBUF)
    wait_scatter(NUM_CHUNKS - 1, (NUM_CHUNKS - 1) % NBUF)
    plsc.subcore_barrier()

    pltpu.sync_copy(acc.at[pl.ds(row0, ROWS_PER_TILE)],
                    out_hbm.at[cid, pl.ds(row0, ROWS_PER_TILE)])

    @pl.when(sid == NUM_SUBCORES - 1)
    def _out_tail():
        pltpu.sync_copy(acc.at[pl.ds(TAIL0, TAIL_ROWS)],
                        out_hbm.at[cid, pl.ds(TAIL0, TAIL_ROWS)])


_layer_call = functools.partial(
    pl.kernel,
    out_type=jax.ShapeDtypeStruct((NUM_CORES, N_NODES, D_FEAT), jnp.float32),
    mesh=plsc.VectorSubcoreMesh(core_axis_name="c", subcore_axis_name="s"),
    scratch_types=[
        pltpu.VMEM((NSLOT, CHUNK), jnp.int32),
        pltpu.VMEM((NSLOT, CHUNK), jnp.int32),
        pltpu.VMEM((NBUF, CHUNK, D_FEAT), jnp.float32),
        pltpu.VMEM_SHARED((ACC_ROWS, D_FEAT), jnp.float32),
        pltpu.SemaphoreType.DMA((NSLOT,)),
        pltpu.SemaphoreType.DMA((NBUF,)),
        pltpu.SemaphoreType.DMA((NBUF,)),
    ],
)(_sc_layer)


def _combine_body(p_ref, o_ref, *, relu):
    s = p_ref[0] + p_ref[1]
    o_ref[...] = jnp.maximum(s, 0.0) if relu else s


def _combine(partials, relu):
    blk = 2000
    return pl.pallas_call(
        functools.partial(_combine_body, relu=relu),
        out_shape=jax.ShapeDtypeStruct((N_NODES, D_FEAT), jnp.float32),
        grid=(N_NODES // blk,),
        in_specs=[pl.BlockSpec((NUM_CORES, blk, D_FEAT), lambda i: (0, i, 0))],
        out_specs=pl.BlockSpec((blk, D_FEAT), lambda i: (i, 0)),
    )(partials)


def kernel(x, edge_index):
    src = edge_index[0].astype(jnp.int32).reshape(NUM_WORKERS, EDGES_PER_WORKER)
    dst = edge_index[1].astype(jnp.int32).reshape(NUM_WORKERS, EDGES_PER_WORKER)
    pad = EPW_PAD - EDGES_PER_WORKER
    src_pad = jnp.pad(src, ((0, 0), (0, pad))).reshape(-1)
    dst_pad = jnp.pad(dst, ((0, 0), (0, pad)),
                      constant_values=TRASH_ROW).reshape(-1)
    zeros = jnp.zeros((N_NODES, D_FEAT), jnp.float32)

    p1 = _layer_call(x, src_pad, dst_pad, zeros)
    h = _combine(p1, relu=True)
    p2 = _layer_call(h, src_pad, dst_pad, zeros)
    return _combine(p2, relu=False)


# speedup vs baseline: 11.7360x; 1.2224x over previous
import functools

import jax
import jax.numpy as jnp
from jax import lax
from jax.experimental import pallas as pl
from jax.experimental.pallas import tpu as pltpu
from jax.experimental.pallas import tpu_sc as plsc

N_NODES = 10000
N_EDGES = 320000
D_FEAT = 128

NUM_CORES = 2
NUM_SUBCORES = 16
NUM_WORKERS = NUM_CORES * NUM_SUBCORES
EDGES_PER_WORKER = N_EDGES // NUM_WORKERS
CHUNK = 56
NUM_CHUNKS = -(-EDGES_PER_WORKER // CHUNK)
EPW_PAD = NUM_CHUNKS * CHUNK
NBUF = 5
TRASH_ROW = N_NODES
ACC_ROWS = N_NODES + 8
ROWS_PER_TILE = 624
TAIL_ROWS = N_NODES - ROWS_PER_TILE * NUM_SUBCORES
TAIL0 = ROWS_PER_TILE * NUM_SUBCORES
NSLOT = 8


def _sc_layer(x_hbm, src_hbm, dst_hbm, zeros_hbm, out_hbm,
              sidx, didx, rows, acc, isem, gsem, ssem):
    cid = lax.axis_index("c")
    sid = lax.axis_index("s")
    wid = sid * NUM_CORES + cid

    row0 = sid * ROWS_PER_TILE
    pltpu.sync_copy(zeros_hbm.at[pl.ds(row0, ROWS_PER_TILE)],
                    acc.at[pl.ds(row0, ROWS_PER_TILE)])

    @pl.when(sid == NUM_SUBCORES - 1)
    def _zero_tail():
        pltpu.sync_copy(zeros_hbm.at[pl.ds(TAIL0, TAIL_ROWS)],
                        acc.at[pl.ds(TAIL0, TAIL_ROWS)])

    plsc.subcore_barrier()

    base = wid * EPW_PAD

    def start_idx(c, s):
        off = base + c * CHUNK
        pltpu.async_copy(src_hbm.at[pl.ds(off, CHUNK)], sidx.at[s], isem.at[s])
        pltpu.async_copy(dst_hbm.at[pl.ds(off, CHUNK)], didx.at[s], isem.at[s])

    def wait_idx(c, s):
        off = base + c * CHUNK
        pltpu.make_async_copy(src_hbm.at[pl.ds(off, CHUNK)], sidx.at[s],
                              isem.at[s]).wait()
        pltpu.make_async_copy(dst_hbm.at[pl.ds(off, CHUNK)], didx.at[s],
                              isem.at[s]).wait()

    def start_gather(c, b):
        s = lax.rem(c, NSLOT)
        pltpu.async_copy(x_hbm.at[sidx.at[s]], rows.at[b], gsem.at[b])

    def wait_gather(c, b):
        s = lax.rem(c, NSLOT)
        pltpu.make_async_copy(x_hbm.at[sidx.at[s]], rows.at[b],
                              gsem.at[b]).wait()

    def start_scatter(c, b):
        s = lax.rem(c, NSLOT)
        pltpu.async_copy(rows.at[b], acc.at[didx.at[s]], ssem.at[b], add=True)

    def wait_scatter(c, b):
        s = lax.rem(c, NSLOT)
        pltpu.make_async_copy(rows.at[b], acc.at[didx.at[s]],
                              ssem.at[b]).wait()

    for c in range(6):
        start_idx(c, c % NSLOT)
    for c in range(3):
        wait_idx(c, c)
        start_gather(c, c)

    def body(c, _):
        b = lax.rem(c, NBUF)

        @pl.when(c >= 2)
        def _drain():
            wait_scatter(c - 2, lax.rem(c - 2, NBUF))

        @pl.when(c + 6 < NUM_CHUNKS)
        def _prefetch():
            start_idx(c + 6, lax.rem(c + 6, NSLOT))

        wait_idx(c + 3, lax.rem(c + 3, NSLOT))
        start_gather(c + 3, lax.rem(c + 3, NBUF))
        wait_gather(c, b)
        start_scatter(c, b)
        return ()

    lax.fori_loop(0, NUM_CHUNKS - 3, body, ())

    for c in range(NUM_CHUNKS - 3, NUM_CHUNKS):
        wait_scatter(c - 2, (c - 2) % NBUF)
        wait_gather(c, c % NBUF)
        start_scatter(c, c % NBUF)
    wait_scatter(NUM_CHUNKS - 2, (NUM_CHUNKS - 2) % NBUF)
    wait_scatter(NUM_CHUNKS - 1, (NUM_CHUNKS - 1) % NBUF)
    plsc.subcore_barrier()

    pltpu.sync_copy(acc.at[pl.ds(row0, ROWS_PER_TILE)],
                    out_hbm.at[cid, pl.ds(row0, ROWS_PER_TILE)])

    @pl.when(sid == NUM_SUBCORES - 1)
    def _out_tail():
        pltpu.sync_copy(acc.at[pl.ds(TAIL0, TAIL_ROWS)],
                        out_hbm.at[cid, pl.ds(TAIL0, TAIL_ROWS)])


_layer_call = functools.partial(
    pl.kernel,
    out_type=jax.ShapeDtypeStruct((NUM_CORES, N_NODES, D_FEAT), jnp.float32),
    mesh=plsc.VectorSubcoreMesh(core_axis_name="c", subcore_axis_name="s"),
    scratch_types=[
        pltpu.VMEM((NSLOT, CHUNK), jnp.int32),
        pltpu.VMEM((NSLOT, CHUNK), jnp.int32),
        pltpu.VMEM((NBUF, CHUNK, D_FEAT), jnp.float32),
        pltpu.VMEM_SHARED((ACC_ROWS, D_FEAT), jnp.float32),
        pltpu.SemaphoreType.DMA((NSLOT,)),
        pltpu.SemaphoreType.DMA((NBUF,)),
        pltpu.SemaphoreType.DMA((NBUF,)),
    ],
)(_sc_layer)


def _combine_body(p_ref, o_ref, *, relu):
    s = p_ref[0] + p_ref[1]
    o_ref[...] = jnp.maximum(s, 0.0) if relu else s


def _combine(partials, relu):
    blk = 2000
    return pl.pallas_call(
        functools.partial(_combine_body, relu=relu),
        out_shape=jax.ShapeDtypeStruct((N_NODES, D_FEAT), jnp.float32),
        grid=(N_NODES // blk,),
        in_specs=[pl.BlockSpec((NUM_CORES, blk, D_FEAT), lambda i: (0, i, 0))],
        out_specs=pl.BlockSpec((blk, D_FEAT), lambda i: (i, 0)),
    )(partials)


def kernel(x, edge_index):
    src = edge_index[0].astype(jnp.int32).reshape(NUM_WORKERS, EDGES_PER_WORKER)
    dst = edge_index[1].astype(jnp.int32).reshape(NUM_WORKERS, EDGES_PER_WORKER)
    pad = EPW_PAD - EDGES_PER_WORKER
    src_pad = jnp.pad(src, ((0, 0), (0, pad))).reshape(-1)
    dst_pad = jnp.pad(dst, ((0, 0), (0, pad)),
                      constant_values=TRASH_ROW).reshape(-1)
    zeros = jnp.zeros((N_NODES, D_FEAT), jnp.float32)

    p1 = _layer_call(x, src_pad, dst_pad, zeros)
    h = _combine(p1, relu=True)
    p2 = _layer_call(h, src_pad, dst_pad, zeros)
    return _combine(p2, relu=False)
